# baseline (speedup 1.0000x reference)
import functools

import jax
import jax.numpy as jnp
from jax import lax
from jax.experimental import pallas as pl
from jax.experimental.pallas import tpu as pltpu
from jax.experimental.pallas import tpu_sc as plsc

B = 4096
L = 50
D_NEWS = 400
CAT_NUM = 18
SUBCAT_NUM = 285
CAT_DIM = 100
SUBCAT_DIM = 100
D_OUT = D_NEWS + CAT_DIM + SUBCAT_DIM

N = B * L
NW = 32
EPW = N // NW
CE = 40
NP = EPW // (2 * CE)

_mesh = plsc.VectorSubcoreMesh(core_axis_name="c", subcore_axis_name="s")


@functools.partial(
    pl.kernel,
    out_type=jax.ShapeDtypeStruct((N, 3, 200), jnp.float32),
    mesh=_mesh,
    scratch_types=[
        pltpu.VMEM((CE,), jnp.int32),
        pltpu.VMEM((CE,), jnp.int32),
        pltpu.VMEM((CE, 1, 256), jnp.float32),
        pltpu.VMEM((CE, 1, 256), jnp.float32),
        pltpu.VMEM((CE, 3, 200), jnp.float32),
        pltpu.VMEM((CE, 3, 200), jnp.float32),
        pltpu.SemaphoreType.DMA,
        pltpu.SemaphoreType.DMA,
        pltpu.SemaphoreType.DMA,
        pltpu.SemaphoreType.DMA,
        pltpu.SemaphoreType.DMA,
        pltpu.SemaphoreType.DMA,
    ],
)
def _sc_assemble(news_hbm, comb_hbm, gidx_hbm, out_hbm,
                 gidx_a, gidx_b, bufc_a, bufc_b, buf3_a, buf3_b,
                 sn_a, sn_b, sg_a, sg_b, sw_a, sw_b):
    wid = lax.axis_index("s") * 2 + lax.axis_index("c")
    ebase0 = wid * EPW

    def assemble(bufc, buf3):
        for i in range(CE):
            for g in range(12):
                buf3[i, 2, pl.ds(g * 16, 16)] = bufc[i, 0, pl.ds(g * 16, 16)]

    def pair(p, carry):
        eb_a = ebase0 + p * 2 * CE
        eb_b = eb_a + CE
        n_a = pltpu.async_copy(news_hbm.at[pl.ds(eb_a, CE)],
                               buf3_a.at[:, 0:2, :], sn_a)
        n_b = pltpu.async_copy(news_hbm.at[pl.ds(eb_b, CE)],
                               buf3_b.at[:, 0:2, :], sn_b)
        pltpu.sync_copy(gidx_hbm.at[pl.ds(eb_a, CE)], gidx_a)
        pltpu.sync_copy(gidx_hbm.at[pl.ds(eb_b, CE)], gidx_b)
        g_a = pltpu.async_copy(comb_hbm.at[gidx_a], bufc_a, sg_a)
        g_b = pltpu.async_copy(comb_hbm.at[gidx_b], bufc_b, sg_b)
        g_a.wait()
        assemble(bufc_a, buf3_a)
        n_a.wait()
        w_a = pltpu.async_copy(buf3_a, out_hbm.at[pl.ds(eb_a, CE)], sw_a)
        g_b.wait()
        assemble(bufc_b, buf3_b)
        n_b.wait()
        w_b = pltpu.async_copy(buf3_b, out_hbm.at[pl.ds(eb_b, CE)], sw_b)
        w_a.wait()
        w_b.wait()
        return carry

    lax.fori_loop(0, NP, pair, 0)


_RB = 64
_PATCH_W = 88
_PATCH_SRC = 12


def _tail_body(out_alias_ref, sub_ref, tab_ref, out_ref):
    del out_alias_ref
    dn = (((0,), (0,)), ((), ()))
    for i in range(_RB):
        oh = (jax.lax.broadcasted_iota(jnp.int32, (SUBCAT_NUM, L), 0)
              == sub_ref[i]).astype(jnp.float32)
        tail = jax.lax.dot_general(
            oh, tab_ref[:, pl.ds(_PATCH_SRC, _PATCH_W)], dn,
            preferred_element_type=jnp.float32)
        out_ref[i, :, 0:_PATCH_W] = tail


def _tail_patch(out3d, sub3, sub_tab):
    return pl.pallas_call(
        _tail_body,
        grid=(B // _RB,),
        in_specs=[
            pl.BlockSpec(memory_space=pl.ANY),
            pl.BlockSpec((_RB, 1, L), lambda i: (i, 0, 0)),
            pl.BlockSpec((SUBCAT_NUM, SUBCAT_DIM), lambda i: (0, 0)),
        ],
        out_specs=pl.BlockSpec((_RB, L, 128), lambda i: (i, 0, 4)),
        out_shape=jax.ShapeDtypeStruct((B, L, D_OUT), jnp.float32),
        input_output_aliases={0: 0},
    )(out3d, sub3, sub_tab)


@jax.jit
def _run(news_representation, category, subCategory, category_embedding,
         subCategory_embedding):
    comb = jnp.concatenate(
        [jnp.broadcast_to(category_embedding[:, None, :],
                          (CAT_NUM, SUBCAT_NUM, CAT_DIM)),
         jnp.broadcast_to(subCategory_embedding[None, :, :],
                          (CAT_NUM, SUBCAT_NUM, SUBCAT_DIM))],
        axis=-1).reshape(CAT_NUM * SUBCAT_NUM, 200)
    comb = jnp.pad(comb, ((0, 0), (0, 56))).reshape(
        CAT_NUM * SUBCAT_NUM, 1, 256)
    news2 = news_representation.reshape(N, 2, 200)
    gidx = (category.astype(jnp.int32) * SUBCAT_NUM
            + subCategory.astype(jnp.int32)).reshape(N)
    out_lin = _sc_assemble(news2, comb, gidx)
    out3d = out_lin.reshape(B, L, D_OUT)
    sub3 = subCategory.astype(jnp.int32).reshape(B, 1, L)
    return _tail_patch(out3d, sub3, subCategory_embedding)


def kernel(news_representation, category, subCategory, category_embedding,
           subCategory_embedding):
    return _run(news_representation, category, subCategory,
                category_embedding, subCategory_embedding)
